# Initial kernel scaffold; baseline (speedup 1.0000x reference)
import functools

import jax
import jax.numpy as jnp
from jax import lax
from jax.experimental import pallas as pl
from jax.experimental.pallas import tpu as pltpu
from jax.experimental.pallas import tpu_sc as plsc

ENT_DIM = 32
REL_DIM = 32
B = 16384
L = 50

_info = plsc.get_sparse_core_info()
NC, NS = _info.num_cores, _info.num_subcores
NW = NC * NS

B_ENT = B * L
ROWS_PER_GATHER = 128
E_ROWS = B_ENT // ROWS_PER_GATHER
E_ROWS_W = E_ROWS // NW
G = 20
R = E_ROWS_W // G
CHUNK = G * ROWS_PER_GATHER

R_ROWS = B // ROWS_PER_GATHER
R_ROWS_W = R_ROWS // NW


def _body(ent_hbm, rel_hbm, eidx_hbm, ridx_hbm, out_e, out_r,
          idx_e_v, idx_r_v, rows_v, rows_r_v, sem):
    wid = lax.axis_index("s") * NC + lax.axis_index("c")

    pltpu.sync_copy(eidx_hbm.at[pl.ds(wid * E_ROWS_W, E_ROWS_W)], idx_e_v)
    pltpu.sync_copy(ridx_hbm.at[pl.ds(wid * R_ROWS_W, R_ROWS_W)], idx_r_v)

    for j in range(R_ROWS_W):
        pltpu.async_copy(rel_hbm.at[idx_r_v.at[j]],
                         rows_r_v.at[pl.ds(j * ROWS_PER_GATHER, ROWS_PER_GATHER)],
                         sem)
    for j in range(R_ROWS_W):
        pltpu.make_async_copy(rel_hbm.at[idx_r_v.at[j]],
                              rows_r_v.at[pl.ds(j * ROWS_PER_GATHER, ROWS_PER_GATHER)],
                              sem).wait()
    pltpu.sync_copy(rows_r_v,
                    out_r.at[pl.ds(wid * (R_ROWS_W * ROWS_PER_GATHER),
                                   R_ROWS_W * ROWS_PER_GATHER)])

    e_base = wid * (E_ROWS_W * ROWS_PER_GATHER)

    def round_body(r, _):
        for j in range(G):
            pltpu.async_copy(ent_hbm.at[idx_e_v.at[r * G + j]],
                             rows_v.at[pl.ds(j * ROWS_PER_GATHER, ROWS_PER_GATHER)],
                             sem)
        for j in range(G):
            pltpu.make_async_copy(ent_hbm.at[idx_e_v.at[r * G + j]],
                                  rows_v.at[pl.ds(j * ROWS_PER_GATHER, ROWS_PER_GATHER)],
                                  sem).wait()
        pltpu.sync_copy(rows_v, out_e.at[pl.ds(e_base + r * CHUNK, CHUNK)])
        return 0

    lax.fori_loop(0, R, round_body, 0)


def _run(entity_table, relation_table, eidx, ridx):
    mesh = plsc.VectorSubcoreMesh(core_axis_name="c", subcore_axis_name="s")
    kern = functools.partial(
        pl.kernel,
        out_type=[
            jax.ShapeDtypeStruct((B_ENT, ENT_DIM), jnp.float32),
            jax.ShapeDtypeStruct((B, REL_DIM), jnp.float32),
        ],
        mesh=mesh,
        scratch_types=[
            pltpu.VMEM((E_ROWS_W, ROWS_PER_GATHER), jnp.int32),
            pltpu.VMEM((R_ROWS_W, ROWS_PER_GATHER), jnp.int32),
            pltpu.VMEM((CHUNK, ENT_DIM), jnp.float32),
            pltpu.VMEM((R_ROWS_W * ROWS_PER_GATHER, REL_DIM), jnp.float32),
            pltpu.SemaphoreType.DMA,
        ],
    )(_body)
    return kern(entity_table, relation_table, eidx, ridx)


def kernel(entity_table, relation_table, entity_idx, relation_idx):
    eidx = entity_idx.reshape(E_ROWS, ROWS_PER_GATHER).astype(jnp.int32)
    ridx = relation_idx.reshape(R_ROWS, ROWS_PER_GATHER).astype(jnp.int32)
    out_e, out_r = _run(entity_table, relation_table, eidx, ridx)
    return out_e.reshape(B, L, ENT_DIM), out_r


# speedup vs baseline: 1.1099x; 1.1099x over previous
import functools

import jax
import jax.numpy as jnp
from jax import lax
from jax.experimental import pallas as pl
from jax.experimental.pallas import tpu as pltpu
from jax.experimental.pallas import tpu_sc as plsc

ENT_DIM = 32
REL_DIM = 32
B = 16384
L = 50

_info = plsc.get_sparse_core_info()
NC, NS = _info.num_cores, _info.num_subcores
NW = NC * NS

B_ENT = B * L
ROWS_PER_GATHER = 128
E_ROWS = B_ENT // ROWS_PER_GATHER
E_ROWS_W = E_ROWS // NW
G = 20
R = E_ROWS_W // G
CHUNK = G * ROWS_PER_GATHER

R_ROWS = B // ROWS_PER_GATHER
R_ROWS_W = R_ROWS // NW


def _body(ent_hbm, rel_hbm, eidx_hbm, ridx_hbm, out_e, out_r,
          idx_e_v, idx_r_v, rows_v, rows_r_v, sem):
    wid = lax.axis_index("s") * NC + lax.axis_index("c")

    pltpu.sync_copy(eidx_hbm.at[pl.ds(wid * E_ROWS_W, E_ROWS_W)], idx_e_v)
    pltpu.sync_copy(ridx_hbm.at[pl.ds(wid * R_ROWS_W, R_ROWS_W)], idx_r_v)

    for j in range(R_ROWS_W):
        pltpu.async_copy(rel_hbm.at[idx_r_v.at[j]],
                         rows_r_v.at[pl.ds(j * ROWS_PER_GATHER, ROWS_PER_GATHER)],
                         sem)
    for j in range(R_ROWS_W):
        pltpu.make_async_copy(rel_hbm.at[idx_r_v.at[j]],
                              rows_r_v.at[pl.ds(j * ROWS_PER_GATHER, ROWS_PER_GATHER)],
                              sem).wait()
    pltpu.sync_copy(rows_r_v,
                    out_r.at[pl.ds(wid * (R_ROWS_W * ROWS_PER_GATHER),
                                   R_ROWS_W * ROWS_PER_GATHER)])

    e_base = wid * (E_ROWS_W * ROWS_PER_GATHER)

    def round_body(r, _):
        for j in range(G):
            pltpu.async_copy(ent_hbm.at[idx_e_v.at[r * G + j]],
                             rows_v.at[pl.ds(j * ROWS_PER_GATHER, ROWS_PER_GATHER)],
                             sem)
        for j in range(G):
            pltpu.make_async_copy(ent_hbm.at[idx_e_v.at[r * G + j]],
                                  rows_v.at[pl.ds(j * ROWS_PER_GATHER, ROWS_PER_GATHER)],
                                  sem).wait()
        pltpu.sync_copy(rows_v, out_e.at[pl.ds(e_base + r * CHUNK, CHUNK)])
        return 0

    lax.fori_loop(0, R, round_body, 0)


def _run(entity_table, relation_table, eidx, ridx):
    mesh = plsc.VectorSubcoreMesh(core_axis_name="c", subcore_axis_name="s")
    kern = functools.partial(
        pl.kernel,
        out_type=[
            jax.ShapeDtypeStruct((B_ENT, ENT_DIM), jnp.float32),
            jax.ShapeDtypeStruct((B, REL_DIM), jnp.float32),
        ],
        mesh=mesh,
        compiler_params=pltpu.CompilerParams(use_tc_tiling_on_sc=False),
        scratch_types=[
            pltpu.VMEM((E_ROWS_W, ROWS_PER_GATHER), jnp.int32),
            pltpu.VMEM((R_ROWS_W, ROWS_PER_GATHER), jnp.int32),
            pltpu.VMEM((CHUNK, ENT_DIM), jnp.float32),
            pltpu.VMEM((R_ROWS_W * ROWS_PER_GATHER, REL_DIM), jnp.float32),
            pltpu.SemaphoreType.DMA,
        ],
    )(_body)
    return kern(entity_table, relation_table, eidx, ridx)


def kernel(entity_table, relation_table, entity_idx, relation_idx):
    eidx = entity_idx.reshape(E_ROWS, ROWS_PER_GATHER).astype(jnp.int32)
    ridx = relation_idx.reshape(R_ROWS, ROWS_PER_GATHER).astype(jnp.int32)
    out_e, out_r = _run(entity_table, relation_table, eidx, ridx)
    return out_e.reshape(B, L, ENT_DIM), out_r


# speedup vs baseline: 1.1135x; 1.0033x over previous
import functools

import jax
import jax.numpy as jnp
from jax import lax
from jax.experimental import pallas as pl
from jax.experimental.pallas import tpu as pltpu
from jax.experimental.pallas import tpu_sc as plsc

ENT_DIM = 32
REL_DIM = 32
B = 16384
L = 50

_info = plsc.get_sparse_core_info()
NC, NS = _info.num_cores, _info.num_subcores
NW = NC * NS

B_ENT = B * L
ROWS_PER_GATHER = 128
E_ROWS = B_ENT // ROWS_PER_GATHER
E_ROWS_W = E_ROWS // NW
G = 10
R = E_ROWS_W // G
CHUNK = G * ROWS_PER_GATHER

R_ROWS = B // ROWS_PER_GATHER
R_ROWS_W = R_ROWS // NW
REL_W = R_ROWS_W * ROWS_PER_GATHER


def _body(ent_hbm, rel_hbm, eidx_hbm, ridx_hbm, out_e, out_r,
          idx_e_v, idx_r_v, rows0_v, rows1_v, rows_r_v,
          semg0, semg1, semo0, semo1, semr):
    wid = lax.axis_index("s") * NC + lax.axis_index("c")
    e_base = wid * (E_ROWS_W * ROWS_PER_GATHER)
    rows_v = (rows0_v, rows1_v)
    semg = (semg0, semg1)
    semo = (semo0, semo1)

    pltpu.sync_copy(eidx_hbm.at[pl.ds(wid * E_ROWS_W, E_ROWS_W)], idx_e_v)
    pltpu.sync_copy(ridx_hbm.at[pl.ds(wid * R_ROWS_W, R_ROWS_W)], idx_r_v)

    def fire(r, b):
        for j in range(G):
            pltpu.async_copy(
                ent_hbm.at[idx_e_v.at[r * G + j]],
                rows_v[b].at[pl.ds(j * ROWS_PER_GATHER, ROWS_PER_GATHER)],
                semg[b])

    def drain(r, b):
        for j in range(G):
            pltpu.make_async_copy(
                ent_hbm.at[idx_e_v.at[r * G + j]],
                rows_v[b].at[pl.ds(j * ROWS_PER_GATHER, ROWS_PER_GATHER)],
                semg[b]).wait()

    def out_start(r, b):
        pltpu.async_copy(rows_v[b], out_e.at[pl.ds(e_base + r * CHUNK, CHUNK)],
                         semo[b])

    def out_wait(r, b):
        pltpu.make_async_copy(rows_v[b],
                              out_e.at[pl.ds(e_base + r * CHUNK, CHUNK)],
                              semo[b]).wait()

    for j in range(R_ROWS_W):
        pltpu.async_copy(rel_hbm.at[idx_r_v.at[j]],
                         rows_r_v.at[pl.ds(j * ROWS_PER_GATHER, ROWS_PER_GATHER)],
                         semr)
    fire(0, 0)
    fire(1, 1)
    for j in range(R_ROWS_W):
        pltpu.make_async_copy(rel_hbm.at[idx_r_v.at[j]],
                              rows_r_v.at[pl.ds(j * ROWS_PER_GATHER, ROWS_PER_GATHER)],
                              semr).wait()
    pltpu.async_copy(rows_r_v, out_r.at[pl.ds(wid * REL_W, REL_W)], semr)

    def round_pair(k, _):
        for b in range(2):
            r = 2 * k + b
            drain(r, b)
            out_start(r, b)
            out_wait(r, b)
            fire(r + 2, b)
        return 0

    lax.fori_loop(0, (R - 2) // 2, round_pair, 0)

    for b in range(2):
        r = R - 2 + b
        drain(r, b)
        out_start(r, b)
    for b in range(2):
        out_wait(R - 2 + b, b)
    pltpu.make_async_copy(rows_r_v, out_r.at[pl.ds(wid * REL_W, REL_W)],
                          semr).wait()


def _run(entity_table, relation_table, eidx, ridx):
    mesh = plsc.VectorSubcoreMesh(core_axis_name="c", subcore_axis_name="s")
    kern = functools.partial(
        pl.kernel,
        out_type=[
            jax.ShapeDtypeStruct((B_ENT, ENT_DIM), jnp.float32),
            jax.ShapeDtypeStruct((B, REL_DIM), jnp.float32),
        ],
        mesh=mesh,
        compiler_params=pltpu.CompilerParams(use_tc_tiling_on_sc=False),
        scratch_types=[
            pltpu.VMEM((E_ROWS_W, ROWS_PER_GATHER), jnp.int32),
            pltpu.VMEM((R_ROWS_W, ROWS_PER_GATHER), jnp.int32),
            pltpu.VMEM((CHUNK, ENT_DIM), jnp.float32),
            pltpu.VMEM((CHUNK, ENT_DIM), jnp.float32),
            pltpu.VMEM((REL_W, REL_DIM), jnp.float32),
            pltpu.SemaphoreType.DMA,
            pltpu.SemaphoreType.DMA,
            pltpu.SemaphoreType.DMA,
            pltpu.SemaphoreType.DMA,
            pltpu.SemaphoreType.DMA,
        ],
    )(_body)
    return kern(entity_table, relation_table, eidx, ridx)


def kernel(entity_table, relation_table, entity_idx, relation_idx):
    eidx = entity_idx.reshape(E_ROWS, ROWS_PER_GATHER).astype(jnp.int32)
    ridx = relation_idx.reshape(R_ROWS, ROWS_PER_GATHER).astype(jnp.int32)
    out_e, out_r = _run(entity_table, relation_table, eidx, ridx)
    return out_e.reshape(B, L, ENT_DIM), out_r


# speedup vs baseline: 1.5462x; 1.3885x over previous
import functools

import jax
import jax.numpy as jnp
from jax import lax
from jax.experimental import pallas as pl
from jax.experimental.pallas import tpu as pltpu
from jax.experimental.pallas import tpu_sc as plsc

ENT_DIM = 32
REL_DIM = 32
B = 16384
L = 50
NREL = 1000

_info = plsc.get_sparse_core_info()
NC, NS = _info.num_cores, _info.num_subcores
NW = NC * NS
BW = B // NW
BTW = BW // 128
NCHUNK = L * BTW
NBUF = 4


def _body(ent_hbm, rel_hbm, eidx_hbm, ridx_hbm, o5, o4,
          idx_v, idxc_v, gb0, gb1, gb2, gb3, tbuf, rtab_v, ridx_v, rtile,
          sg0, sg1, sg2, sg3, sw, srt):
    wid = lax.axis_index("s") * NC + lax.axis_index("c")
    b0 = wid * BW
    gbufs = (gb0, gb1, gb2, gb3)
    sgs = (sg0, sg1, sg2, sg3)
    iota = lax.iota(jnp.int32, 16)

    pltpu.async_copy(rel_hbm, rtab_v, srt)
    pltpu.sync_copy(eidx_hbm.at[pl.ds(b0, BW)], idx_v)
    pltpu.sync_copy(ridx_hbm.at[pl.ds(b0, BW)], ridx_v)

    def repack_and_fire(c, s):
        k = c // L
        l = c % L
        lvec = jnp.full((16,), 0, jnp.int32) + l
        for t in range(8):
            rows = plsc.load_gather(idx_v, [k * 128 + 16 * t + iota, lvec])
            idxc_v[s, pl.ds(16 * t, 16)] = rows
        pltpu.async_copy(ent_hbm.at[idxc_v.at[s]], gbufs[s], sgs[s])

    def process(c, s):
        k = c // L
        l = c % L
        pltpu.make_async_copy(ent_hbm.at[idxc_v.at[s]], gbufs[s],
                              sgs[s]).wait()

        def tr_row(j, _):
            jvec = jnp.full((16,), 0, jnp.int32) + j
            for t in range(8):
                v = plsc.load_gather(gbufs[s], [16 * t + iota, jvec])
                tbuf[j, pl.ds(16 * t, 16)] = v
            return 0

        lax.fori_loop(0, ENT_DIM, tr_row, 0)
        for jt in range(4):
            pltpu.async_copy(tbuf.at[pl.ds(8 * jt, 8)],
                             o5.at[l, jt, wid * BTW + k], sw)
        for jt in range(4):
            pltpu.make_async_copy(tbuf.at[pl.ds(8 * jt, 8)],
                                  o5.at[l, jt, wid * BTW + k], sw).wait()

    for c in range(NBUF - 1):
        repack_and_fire(c, c)

    def step(kk, _):
        for s in range(NBUF):
            c = kk * NBUF + s
            process(c, s)
            cf = c + NBUF - 1

            @pl.when(cf < NCHUNK)
            def _():
                repack_and_fire(cf, (s + NBUF - 1) % NBUF)
        return 0

    lax.fori_loop(0, NCHUNK // NBUF, step, 0)

    pltpu.make_async_copy(rel_hbm, rtab_v, srt).wait()
    for k in range(BTW):
        for jt in range(4):

            def rel_row(jr, _, _k=k, _jt=jt):
                jvec = jnp.full((16,), 0, jnp.int32) + (8 * _jt + jr)
                for t in range(8):
                    idxv = ridx_v[pl.ds(128 * _k + 16 * t, 16)]
                    v = plsc.load_gather(rtab_v, [idxv, jvec])
                    rtile[jr, pl.ds(16 * t, 16)] = v
                return 0

            lax.fori_loop(0, 8, rel_row, 0)
            pltpu.sync_copy(rtile, o4.at[jt, wid * BTW + k])


def _run(entity_table, relation_table, entity_idx, relation_idx):
    mesh = plsc.VectorSubcoreMesh(core_axis_name="c", subcore_axis_name="s")
    kern = functools.partial(
        pl.kernel,
        out_type=[
            jax.ShapeDtypeStruct((L, 4, B // 128, 8, 128), jnp.float32),
            jax.ShapeDtypeStruct((4, B // 128, 8, 128), jnp.float32),
        ],
        mesh=mesh,
        compiler_params=pltpu.CompilerParams(
            use_tc_tiling_on_sc=False, needs_layout_passes=False),
        scratch_types=[
            pltpu.VMEM((BW, L), jnp.int32),
            pltpu.VMEM((NBUF, 128), jnp.int32),
            pltpu.VMEM((128, ENT_DIM), jnp.float32),
            pltpu.VMEM((128, ENT_DIM), jnp.float32),
            pltpu.VMEM((128, ENT_DIM), jnp.float32),
            pltpu.VMEM((128, ENT_DIM), jnp.float32),
            pltpu.VMEM((ENT_DIM, 128), jnp.float32),
            pltpu.VMEM((NREL, REL_DIM), jnp.float32),
            pltpu.VMEM((BW,), jnp.int32),
            pltpu.VMEM((8, 128), jnp.float32),
            pltpu.SemaphoreType.DMA,
            pltpu.SemaphoreType.DMA,
            pltpu.SemaphoreType.DMA,
            pltpu.SemaphoreType.DMA,
            pltpu.SemaphoreType.DMA,
            pltpu.SemaphoreType.DMA,
        ],
    )(_body)
    return kern(entity_table, relation_table, entity_idx, relation_idx)


def kernel(entity_table, relation_table, entity_idx, relation_idx):
    eidx = entity_idx.astype(jnp.int32)
    ridx = relation_idx.astype(jnp.int32)
    out5, out_r4 = _run(entity_table, relation_table, eidx, ridx)
    out_e = out5.transpose(2, 4, 0, 1, 3).reshape(B, L, ENT_DIM)
    out_r = out_r4.transpose(1, 3, 0, 2).reshape(B, REL_DIM)
    return out_e, out_r
